# baseline (speedup 1.0000x reference)
import jax
import jax.numpy as jnp
from jax import lax
from jax.experimental import pallas as pl
from jax.experimental.pallas import tpu as pltpu
from jax.experimental.pallas import tpu_sc as plsc

T = 5
S1 = 5
S2 = 2
D = 128
H1 = 64
H2 = 32
NP = 10240
NC, NS = 2, 16
NW = NC * NS
CHUNK = 384
NB = T * S2 * S1 + T * S1 + 1
NBP = 78
N_ROWS = NBP * NP
N_CHUNKS = N_ROWS // CHUNK
PER_W = N_CHUNKS // NW
IR = CHUNK // 128


def _sc_gather_body(x_hbm, idx_hbm, g_hbm,
                    idx_a, rows_a, idx_b, rows_b, sem_ga, sem_gb):
    wid = lax.axis_index("s") * NC + lax.axis_index("c")

    def copy_idx(c, idx_v):
        pltpu.sync_copy(idx_hbm.at[pl.ds(c * IR, IR)], idx_v)

    def fire_gather(idx_v, rows_v, sem_g):
        for j in range(IR):
            pltpu.async_copy(x_hbm.at[idx_v.at[j]],
                             rows_v.at[pl.ds(j * 128, 128)], sem_g)

    def drain_gather(rows_v, sem_g):
        pltpu.make_async_copy(x_hbm.at[pl.ds(0, CHUNK)], rows_v, sem_g).wait()

    def wb(c, rows_v):
        pltpu.sync_copy(rows_v, g_hbm.at[pl.ds(c * CHUNK, CHUNK)])

    def ch(i):
        return wid + i * NW

    copy_idx(ch(0), idx_a)
    fire_gather(idx_a, rows_a, sem_ga)

    def body(k, _):
        copy_idx(ch(2 * k + 1), idx_b)
        fire_gather(idx_b, rows_b, sem_gb)
        drain_gather(rows_a, sem_ga)
        wb(ch(2 * k), rows_a)
        copy_idx(ch(2 * k + 2), idx_a)
        fire_gather(idx_a, rows_a, sem_ga)
        drain_gather(rows_b, sem_gb)
        wb(ch(2 * k + 1), rows_b)
        return 0

    lax.fori_loop(0, (PER_W - 1) // 2, body, 0)
    drain_gather(rows_a, sem_ga)
    wb(ch(PER_W - 1), rows_a)


def _sc_gather(x, idx2d):
    mesh = plsc.VectorSubcoreMesh(core_axis_name="c", subcore_axis_name="s",
                                  num_cores=NC, num_subcores=NS)
    return pl.kernel(
        _sc_gather_body,
        out_type=jax.ShapeDtypeStruct((N_ROWS, D), jnp.float32),
        mesh=mesh,
        compiler_params=pltpu.CompilerParams(use_tc_tiling_on_sc=False),
        scratch_types=[
            pltpu.VMEM((IR, 128), jnp.int32),
            pltpu.VMEM((CHUNK, D), jnp.float32),
            pltpu.VMEM((IR, 128), jnp.int32),
            pltpu.VMEM((CHUNK, D), jnp.float32),
            pltpu.SemaphoreType.DMA,
            pltpu.SemaphoreType.DMA,
        ],
    )(x, idx2d)


def _head_body(h2_ref, h1_ref, h0_ref, ws0_ref, wn0_ref, b0_ref,
               ws1_ref, wn1_ref, b1_ref, wp_ref, bp_ref, out_ref):
    blk = out_ref.shape[0]
    f32 = jnp.float32
    ws0 = ws0_ref[...]
    wn0 = wn0_ref[...]
    b0 = b0_ref[...]
    top_self = jnp.dot(h0_ref[0], ws0, preferred_element_type=f32)
    acc = jnp.broadcast_to(bp_ref[...], (blk, D))
    for t in range(T):
        m = h1_ref[t * S1]
        for j in range(1, S1):
            m = m + h1_ref[t * S1 + j]
        mean5 = m / 5.0
        top = top_self + jnp.dot(mean5, wn0, preferred_element_type=f32) + b0
        s0_top = (top >= 1.0).astype(f32)
        m1 = jnp.zeros((blk, H1), f32)
        for j in range(S1):
            mean2 = (h2_ref[(t * S2 + 0) * S1 + j]
                     + h2_ref[(t * S2 + 1) * S1 + j]) / 2.0
            bot = (jnp.dot(h1_ref[t * S1 + j], ws0, preferred_element_type=f32)
                   + jnp.dot(mean2, wn0, preferred_element_type=f32) + b0)
            m1 = m1 + (bot >= 1.0).astype(f32)
        neigh1 = m1 / 5.0
        out1 = (jnp.dot(s0_top, ws1_ref[...], preferred_element_type=f32)
                + jnp.dot(neigh1, wn1_ref[...], preferred_element_type=f32)
                + b1_ref[...])
        s1 = (out1 >= 1.0).astype(f32)
        acc = acc + jnp.dot(s1, wp_ref[t * H2:(t + 1) * H2, :],
                            preferred_element_type=f32)
    out_ref[...] = acc


def _head(g, ws0, wn0, b0, ws1, wn1, b1, wp, bp):
    blk = 256
    grid = NP // blk
    return pl.pallas_call(
        _head_body,
        grid=(grid,),
        in_specs=[
            pl.BlockSpec((T * S2 * S1, blk, D), lambda i: (0, i, 0)),
            pl.BlockSpec((T * S1, blk, D), lambda i: (2, i, 0)),
            pl.BlockSpec((1, blk, D), lambda i: (NB - 1, i, 0)),
            pl.BlockSpec((D, H1), lambda i: (0, 0)),
            pl.BlockSpec((D, H1), lambda i: (0, 0)),
            pl.BlockSpec((1, H1), lambda i: (0, 0)),
            pl.BlockSpec((H1, H2), lambda i: (0, 0)),
            pl.BlockSpec((H1, H2), lambda i: (0, 0)),
            pl.BlockSpec((1, H2), lambda i: (0, 0)),
            pl.BlockSpec((T * H2, D), lambda i: (0, 0)),
            pl.BlockSpec((1, D), lambda i: (0, 0)),
        ],
        out_specs=pl.BlockSpec((blk, D), lambda i: (i, 0)),
        out_shape=jax.ShapeDtypeStruct((NP, D), jnp.float32),
    )(g, g, g, ws0, wn0, b0, ws1, wn1, b1, wp, bp)


def kernel(x, nodes, nbr1, nbr2, W_self0, W_neigh0, b0, W_self1, W_neigh1,
           b1, Wp, bp):
    n = nodes.shape[0]

    idx2 = jnp.zeros((T, S2, S1, NP), jnp.int32)
    idx2 = idx2.at[:, :, :, :n].set(
        nbr2.reshape(T, n, S1, S2).transpose(0, 3, 2, 1))
    idx1 = jnp.zeros((T, S1, NP), jnp.int32)
    idx1 = idx1.at[:, :, :n].set(nbr1.reshape(T, n, S1).transpose(0, 2, 1))
    nodes_pad = jnp.zeros((1, NP), jnp.int32).at[0, :n].set(nodes)
    idx_flat = jnp.concatenate(
        [idx2.reshape(T * S2 * S1, NP), idx1.reshape(T * S1, NP), nodes_pad,
         jnp.zeros((NBP - NB, NP), jnp.int32)],
        axis=0)

    g = _sc_gather(x, idx_flat.reshape(N_ROWS // 128, 128))
    out = _head(g.reshape(NBP, NP, D), W_self0, W_neigh0, b0[None, :],
                W_self1, W_neigh1, b1[None, :], Wp, bp[None, :])
    return out[:n]


# speedup vs baseline: 1.5222x; 1.5222x over previous
import jax
import jax.numpy as jnp
from jax import lax
from jax.experimental import pallas as pl
from jax.experimental.pallas import tpu as pltpu
from jax.experimental.pallas import tpu_sc as plsc

T = 5
S1 = 5
S2 = 2
D = 128
H1 = 64
H2 = 32
NP = 10240
NC, NS = 2, 16
NW = NC * NS
NB = T * S2 * S1 + T * S1 + 1
N_ROWS = NB * NP
UNIT = 128
PER_W = N_ROWS // (NW * UNIT)
RING = 5


def _sc_gather_body(x_hbm, idx_hbm, g_hbm, idx_buf, *bufs_and_sems):
    bufs = bufs_and_sems[:RING]
    sems = bufs_and_sems[RING:]
    wid = lax.axis_index("s") * NC + lax.axis_index("c")
    base = wid * PER_W

    pltpu.sync_copy(idx_hbm.at[wid], idx_buf)

    def fire(u, r):
        pltpu.async_copy(x_hbm.at[idx_buf.at[u]], bufs[r], sems[r])

    def drain(r):
        pltpu.make_async_copy(x_hbm.at[pl.ds(0, UNIT)], bufs[r],
                              sems[r]).wait()

    for r in range(RING - 1):
        fire(r, r)

    def body(k, _):
        for r in range(RING):
            u = k * RING + r
            drain(r)
            pltpu.sync_copy(bufs[r], g_hbm.at[pl.ds((base + u) * UNIT, UNIT)])
            nxt = u + RING - 1

            @pl.when(nxt < PER_W)
            def _():
                fire(nxt, (r + RING - 1) % RING)
        return 0

    lax.fori_loop(0, PER_W // RING, body, 0)


def _sc_gather(x, idx3d):
    mesh = plsc.VectorSubcoreMesh(core_axis_name="c", subcore_axis_name="s",
                                  num_cores=NC, num_subcores=NS)
    return pl.kernel(
        _sc_gather_body,
        out_type=jax.ShapeDtypeStruct((N_ROWS, D), jnp.float32),
        mesh=mesh,
        compiler_params=pltpu.CompilerParams(use_tc_tiling_on_sc=False),
        scratch_types=(
            [pltpu.VMEM((PER_W, 128), jnp.int32)]
            + [pltpu.VMEM((UNIT, D), jnp.float32)] * RING
            + [pltpu.SemaphoreType.DMA] * RING
        ),
    )(x, idx3d)


def _head_body(h2_ref, h1_ref, h0_ref, ws0_ref, wn0_ref, b0_ref,
               ws1_ref, wn1_ref, b1_ref, wp_ref, bp_ref, out_ref):
    blk = out_ref.shape[0]
    f32 = jnp.float32
    ws0 = ws0_ref[...]
    wn0 = wn0_ref[...]
    b0 = b0_ref[...]
    top_self = jnp.dot(h0_ref[0], ws0, preferred_element_type=f32)
    acc = jnp.broadcast_to(bp_ref[...], (blk, D))
    for t in range(T):
        m = h1_ref[t * S1]
        for j in range(1, S1):
            m = m + h1_ref[t * S1 + j]
        mean5 = m / 5.0
        top = top_self + jnp.dot(mean5, wn0, preferred_element_type=f32) + b0
        s0_top = (top >= 1.0).astype(f32)
        m1 = jnp.zeros((blk, H1), f32)
        for j in range(S1):
            mean2 = (h2_ref[(t * S2 + 0) * S1 + j]
                     + h2_ref[(t * S2 + 1) * S1 + j]) / 2.0
            bot = (jnp.dot(h1_ref[t * S1 + j], ws0, preferred_element_type=f32)
                   + jnp.dot(mean2, wn0, preferred_element_type=f32) + b0)
            m1 = m1 + (bot >= 1.0).astype(f32)
        neigh1 = m1 / 5.0
        out1 = (jnp.dot(s0_top, ws1_ref[...], preferred_element_type=f32)
                + jnp.dot(neigh1, wn1_ref[...], preferred_element_type=f32)
                + b1_ref[...])
        s1 = (out1 >= 1.0).astype(f32)
        acc = acc + jnp.dot(s1, wp_ref[t * H2:(t + 1) * H2, :],
                            preferred_element_type=f32)
    out_ref[...] = acc


def _head(g, ws0, wn0, b0, ws1, wn1, b1, wp, bp):
    blk = 256
    grid = NP // blk
    return pl.pallas_call(
        _head_body,
        grid=(grid,),
        in_specs=[
            pl.BlockSpec((T * S2 * S1, blk, D), lambda i: (0, i, 0)),
            pl.BlockSpec((T * S1, blk, D), lambda i: (2, i, 0)),
            pl.BlockSpec((1, blk, D), lambda i: (NB - 1, i, 0)),
            pl.BlockSpec((D, H1), lambda i: (0, 0)),
            pl.BlockSpec((D, H1), lambda i: (0, 0)),
            pl.BlockSpec((1, H1), lambda i: (0, 0)),
            pl.BlockSpec((H1, H2), lambda i: (0, 0)),
            pl.BlockSpec((H1, H2), lambda i: (0, 0)),
            pl.BlockSpec((1, H2), lambda i: (0, 0)),
            pl.BlockSpec((T * H2, D), lambda i: (0, 0)),
            pl.BlockSpec((1, D), lambda i: (0, 0)),
        ],
        out_specs=pl.BlockSpec((blk, D), lambda i: (i, 0)),
        out_shape=jax.ShapeDtypeStruct((NP, D), jnp.float32),
    )(g, g, g, ws0, wn0, b0, ws1, wn1, b1, wp, bp)


def kernel(x, nodes, nbr1, nbr2, W_self0, W_neigh0, b0, W_self1, W_neigh1,
           b1, Wp, bp):
    n = nodes.shape[0]

    idx2 = jnp.zeros((T, S2, S1, NP), jnp.int32)
    idx2 = idx2.at[:, :, :, :n].set(
        nbr2.reshape(T, n, S1, S2).transpose(0, 3, 2, 1))
    idx1 = jnp.zeros((T, S1, NP), jnp.int32)
    idx1 = idx1.at[:, :, :n].set(nbr1.reshape(T, n, S1).transpose(0, 2, 1))
    nodes_pad = jnp.zeros((1, NP), jnp.int32).at[0, :n].set(nodes)
    idx_flat = jnp.concatenate(
        [idx2.reshape(T * S2 * S1, NP), idx1.reshape(T * S1, NP), nodes_pad],
        axis=0)

    g = _sc_gather(x, idx_flat.reshape(NW, PER_W, 128))
    out = _head(g.reshape(NB, NP, D), W_self0, W_neigh0, b0[None, :],
                W_self1, W_neigh1, b1[None, :], Wp, bp[None, :])
    return out[:n]


# speedup vs baseline: 1.5467x; 1.0161x over previous
import jax
import jax.numpy as jnp
from jax import lax
from jax.experimental import pallas as pl
from jax.experimental.pallas import tpu as pltpu
from jax.experimental.pallas import tpu_sc as plsc

T = 5
S1 = 5
S2 = 2
D = 128
H1 = 64
H2 = 32
NP = 10240
NC, NS = 2, 16
NW = NC * NS
NB = T * S2 * S1 + T * S1 + 1
N_ROWS = NB * NP
UNIT = 256
ROWS_W = N_ROWS // NW
PER_W = ROWS_W // UNIT
RING = 3


def _sc_gather_body(x_hbm, idx_hbm, g_hbm, idx_buf, *bufs_and_sems):
    bufs = bufs_and_sems[:RING]
    sems = bufs_and_sems[RING:]
    wid = lax.axis_index("s") * NC + lax.axis_index("c")
    base = wid * ROWS_W

    pltpu.sync_copy(idx_hbm.at[wid], idx_buf)

    def fire(u, r):
        pltpu.async_copy(x_hbm.at[idx_buf.at[pl.ds(u * UNIT, UNIT)]],
                         bufs[r], sems[r])

    def drain(r):
        pltpu.make_async_copy(x_hbm.at[pl.ds(0, UNIT)], bufs[r],
                              sems[r]).wait()

    for r in range(RING - 1):
        fire(r, r)

    def body(k, _):
        for r in range(RING):
            u = k * RING + r
            drain(r)
            pltpu.sync_copy(bufs[r], g_hbm.at[pl.ds(base + u * UNIT, UNIT)])
            nxt = u + RING - 1

            @pl.when(nxt < PER_W)
            def _():
                fire(nxt, (r + RING - 1) % RING)
        return 0

    lax.fori_loop(0, PER_W // RING, body, 0)
    for e in range(PER_W % RING):
        u = (PER_W // RING) * RING + e
        drain(e)
        pltpu.sync_copy(bufs[e], g_hbm.at[pl.ds(base + u * UNIT, UNIT)])


def _sc_gather(x, idx2d):
    mesh = plsc.VectorSubcoreMesh(core_axis_name="c", subcore_axis_name="s",
                                  num_cores=NC, num_subcores=NS)
    return pl.kernel(
        _sc_gather_body,
        out_type=jax.ShapeDtypeStruct((N_ROWS, D), jnp.float32),
        mesh=mesh,
        compiler_params=pltpu.CompilerParams(use_tc_tiling_on_sc=False),
        scratch_types=(
            [pltpu.VMEM((ROWS_W,), jnp.int32)]
            + [pltpu.VMEM((UNIT, D), jnp.float32)] * RING
            + [pltpu.SemaphoreType.DMA] * RING
        ),
    )(x, idx2d)


def _head_body(h2_ref, h1_ref, h0_ref, ws0_ref, wn0_ref, b0_ref,
               ws1_ref, wn1_ref, b1_ref, wp_ref, bp_ref, out_ref):
    blk = out_ref.shape[0]
    f32 = jnp.float32
    ws0 = ws0_ref[...]
    wn0 = wn0_ref[...]
    b0 = b0_ref[...]
    top_self = jnp.dot(h0_ref[0], ws0, preferred_element_type=f32)
    acc = jnp.broadcast_to(bp_ref[...], (blk, D))
    for t in range(T):
        m = h1_ref[t * S1]
        for j in range(1, S1):
            m = m + h1_ref[t * S1 + j]
        mean5 = m / 5.0
        top = top_self + jnp.dot(mean5, wn0, preferred_element_type=f32) + b0
        s0_top = (top >= 1.0).astype(f32)
        m1 = jnp.zeros((blk, H1), f32)
        for j in range(S1):
            mean2 = (h2_ref[(t * S2 + 0) * S1 + j]
                     + h2_ref[(t * S2 + 1) * S1 + j]) / 2.0
            bot = (jnp.dot(h1_ref[t * S1 + j], ws0, preferred_element_type=f32)
                   + jnp.dot(mean2, wn0, preferred_element_type=f32) + b0)
            m1 = m1 + (bot >= 1.0).astype(f32)
        neigh1 = m1 / 5.0
        out1 = (jnp.dot(s0_top, ws1_ref[...], preferred_element_type=f32)
                + jnp.dot(neigh1, wn1_ref[...], preferred_element_type=f32)
                + b1_ref[...])
        s1 = (out1 >= 1.0).astype(f32)
        acc = acc + jnp.dot(s1, wp_ref[t * H2:(t + 1) * H2, :],
                            preferred_element_type=f32)
    out_ref[...] = acc


def _head(g, ws0, wn0, b0, ws1, wn1, b1, wp, bp):
    blk = 256
    grid = NP // blk
    return pl.pallas_call(
        _head_body,
        grid=(grid,),
        in_specs=[
            pl.BlockSpec((T * S2 * S1, blk, D), lambda i: (0, i, 0)),
            pl.BlockSpec((T * S1, blk, D), lambda i: (2, i, 0)),
            pl.BlockSpec((1, blk, D), lambda i: (NB - 1, i, 0)),
            pl.BlockSpec((D, H1), lambda i: (0, 0)),
            pl.BlockSpec((D, H1), lambda i: (0, 0)),
            pl.BlockSpec((1, H1), lambda i: (0, 0)),
            pl.BlockSpec((H1, H2), lambda i: (0, 0)),
            pl.BlockSpec((H1, H2), lambda i: (0, 0)),
            pl.BlockSpec((1, H2), lambda i: (0, 0)),
            pl.BlockSpec((T * H2, D), lambda i: (0, 0)),
            pl.BlockSpec((1, D), lambda i: (0, 0)),
        ],
        out_specs=pl.BlockSpec((blk, D), lambda i: (i, 0)),
        out_shape=jax.ShapeDtypeStruct((NP, D), jnp.float32),
    )(g, g, g, ws0, wn0, b0, ws1, wn1, b1, wp, bp)


def kernel(x, nodes, nbr1, nbr2, W_self0, W_neigh0, b0, W_self1, W_neigh1,
           b1, Wp, bp):
    n = nodes.shape[0]

    idx2 = jnp.zeros((T, S2, S1, NP), jnp.int32)
    idx2 = idx2.at[:, :, :, :n].set(
        nbr2.reshape(T, n, S1, S2).transpose(0, 3, 2, 1))
    idx1 = jnp.zeros((T, S1, NP), jnp.int32)
    idx1 = idx1.at[:, :, :n].set(nbr1.reshape(T, n, S1).transpose(0, 2, 1))
    nodes_pad = jnp.zeros((1, NP), jnp.int32).at[0, :n].set(nodes)
    idx_flat = jnp.concatenate(
        [idx2.reshape(T * S2 * S1, NP), idx1.reshape(T * S1, NP), nodes_pad],
        axis=0)

    g = _sc_gather(x, idx_flat.reshape(NW, ROWS_W))
    out = _head(g.reshape(NB, NP, D), W_self0, W_neigh0, b0[None, :],
                W_self1, W_neigh1, b1[None, :], Wp, bp[None, :])
    return out[:n]


# speedup vs baseline: 1.6612x; 1.0740x over previous
import jax
import jax.numpy as jnp
from jax import lax
from jax.experimental import pallas as pl
from jax.experimental.pallas import tpu as pltpu
from jax.experimental.pallas import tpu_sc as plsc

T = 5
S1 = 5
S2 = 2
D = 128
H1 = 64
H2 = 32
NP = 10240
NPH = 5120
NC, NS = 2, 16
NW = NC * NS
NB = T * S2 * S1 + T * S1 + 1
N_ROWS = NB * NPH
UNIT = 128
ROWS_W = N_ROWS // NW
PER_W = ROWS_W // UNIT
RING = 5


def _sc_gather_body(x_hbm, idx_hbm, g_hbm, idx_buf, *bufs_and_sems):
    bufs = bufs_and_sems[:RING]
    sems = bufs_and_sems[RING:]
    wid = lax.axis_index("s") * NC + lax.axis_index("c")
    base = wid * ROWS_W

    pltpu.sync_copy(idx_hbm.at[wid], idx_buf)

    def fire(u, r):
        pltpu.async_copy(x_hbm.at[idx_buf.at[pl.ds(u * UNIT, UNIT)]],
                         bufs[r], sems[r])

    def drain(r):
        pltpu.make_async_copy(x_hbm.at[pl.ds(0, UNIT)], bufs[r],
                              sems[r]).wait()

    for r in range(RING - 1):
        fire(r, r)

    def body(k, _):
        for r in range(RING):
            u = k * RING + r
            drain(r)
            pltpu.sync_copy(bufs[r], g_hbm.at[pl.ds(base + u * UNIT, UNIT)])
            nxt = u + RING - 1

            @pl.when(nxt < PER_W)
            def _():
                fire(nxt, (r + RING - 1) % RING)
        return 0

    lax.fori_loop(0, PER_W // RING, body, 0)
    for e in range(PER_W % RING):
        u = (PER_W // RING) * RING + e
        drain(e)
        pltpu.sync_copy(bufs[e], g_hbm.at[pl.ds(base + u * UNIT, UNIT)])


def _sc_gather(x, idx2d):
    mesh = plsc.VectorSubcoreMesh(core_axis_name="c", subcore_axis_name="s",
                                  num_cores=NC, num_subcores=NS)
    return pl.kernel(
        _sc_gather_body,
        out_type=jax.ShapeDtypeStruct((N_ROWS, D), jnp.float32),
        mesh=mesh,
        compiler_params=pltpu.CompilerParams(use_tc_tiling_on_sc=False),
        scratch_types=(
            [pltpu.VMEM((ROWS_W,), jnp.int32)]
            + [pltpu.VMEM((UNIT, D), jnp.float32)] * RING
            + [pltpu.SemaphoreType.DMA] * RING
        ),
    )(x, idx2d)


def _head_body(h2_ref, h1_ref, h0_ref, ws0_ref, wn0_ref, b0_ref,
               ws1_ref, wn1_ref, b1_ref, wp_ref, bp_ref, out_ref):
    blk = out_ref.shape[0]
    f32 = jnp.float32
    ws0 = ws0_ref[...]
    wn0 = wn0_ref[...]
    b0 = b0_ref[...]
    top_self = jnp.dot(h0_ref[0], ws0, preferred_element_type=f32)
    acc = jnp.broadcast_to(bp_ref[...], (blk, D))
    for t in range(T):
        m = h1_ref[t * S1]
        for j in range(1, S1):
            m = m + h1_ref[t * S1 + j]
        mean5 = m / 5.0
        top = top_self + jnp.dot(mean5, wn0, preferred_element_type=f32) + b0
        s0_top = (top >= 1.0).astype(f32)
        m1 = jnp.zeros((blk, H1), f32)
        for j in range(S1):
            mean2 = (h2_ref[(t * S2 + 0) * S1 + j]
                     + h2_ref[(t * S2 + 1) * S1 + j]) / 2.0
            bot = (jnp.dot(h1_ref[t * S1 + j], ws0, preferred_element_type=f32)
                   + jnp.dot(mean2, wn0, preferred_element_type=f32) + b0)
            m1 = m1 + (bot >= 1.0).astype(f32)
        neigh1 = m1 / 5.0
        out1 = (jnp.dot(s0_top, ws1_ref[...], preferred_element_type=f32)
                + jnp.dot(neigh1, wn1_ref[...], preferred_element_type=f32)
                + b1_ref[...])
        s1 = (out1 >= 1.0).astype(f32)
        acc = acc + jnp.dot(s1, wp_ref[t * H2:(t + 1) * H2, :],
                            preferred_element_type=f32)
    out_ref[...] = acc


def _head(g, ws0, wn0, b0, ws1, wn1, b1, wp, bp):
    blk = 256
    grid = NPH // blk
    return pl.pallas_call(
        _head_body,
        grid=(grid,),
        in_specs=[
            pl.BlockSpec((T * S2 * S1, blk, D), lambda i: (0, i, 0)),
            pl.BlockSpec((T * S1, blk, D), lambda i: (2, i, 0)),
            pl.BlockSpec((1, blk, D), lambda i: (NB - 1, i, 0)),
            pl.BlockSpec((D, H1), lambda i: (0, 0)),
            pl.BlockSpec((D, H1), lambda i: (0, 0)),
            pl.BlockSpec((1, H1), lambda i: (0, 0)),
            pl.BlockSpec((H1, H2), lambda i: (0, 0)),
            pl.BlockSpec((H1, H2), lambda i: (0, 0)),
            pl.BlockSpec((1, H2), lambda i: (0, 0)),
            pl.BlockSpec((T * H2, D), lambda i: (0, 0)),
            pl.BlockSpec((1, D), lambda i: (0, 0)),
        ],
        out_specs=pl.BlockSpec((blk, D), lambda i: (i, 0)),
        out_shape=jax.ShapeDtypeStruct((NPH, D), jnp.float32),
    )(g, g, g, ws0, wn0, b0, ws1, wn1, b1, wp, bp)


def kernel(x, nodes, nbr1, nbr2, W_self0, W_neigh0, b0, W_self1, W_neigh1,
           b1, Wp, bp):
    n = nodes.shape[0]

    idx2 = jnp.zeros((T, S2, S1, NP), jnp.int32)
    idx2 = idx2.at[:, :, :, :n].set(
        nbr2.reshape(T, n, S1, S2).transpose(0, 3, 2, 1))
    idx1 = jnp.zeros((T, S1, NP), jnp.int32)
    idx1 = idx1.at[:, :, :n].set(nbr1.reshape(T, n, S1).transpose(0, 2, 1))
    nodes_pad = jnp.zeros((1, NP), jnp.int32).at[0, :n].set(nodes)
    idx_flat = jnp.concatenate(
        [idx2.reshape(T * S2 * S1, NP), idx1.reshape(T * S1, NP), nodes_pad],
        axis=0)

    outs = []
    for p in range(NP // NPH):
        idx_p = idx_flat[:, p * NPH:(p + 1) * NPH].reshape(NW, ROWS_W)
        g = _sc_gather(x, idx_p)
        outs.append(_head(g.reshape(NB, NPH, D), W_self0, W_neigh0,
                          b0[None, :], W_self1, W_neigh1, b1[None, :],
                          Wp, bp[None, :]))
    return jnp.concatenate(outs, axis=0)[:n]


# speedup vs baseline: 1.8219x; 1.0967x over previous
import jax
import jax.numpy as jnp
from jax import lax
from jax.experimental import pallas as pl
from jax.experimental.pallas import tpu as pltpu
from jax.experimental.pallas import tpu_sc as plsc

T = 5
S1 = 5
S2 = 2
D = 128
H1 = 64
H2 = 32
NP = 10240
PARTS = (5120, 2560, 1280, 1280)
NC, NS = 2, 16
NW = NC * NS
NB = T * S2 * S1 + T * S1 + 1
RING = 5


def _sc_gather(x, idx2d, nph, unit):
    rows_w = NB * nph // NW
    per_w = rows_w // unit

    def body(x_hbm, idx_hbm, g_hbm, idx_buf, *bufs_and_sems):
        bufs = bufs_and_sems[:RING]
        sems = bufs_and_sems[RING:]
        wid = lax.axis_index("s") * NC + lax.axis_index("c")
        base = wid * rows_w

        pltpu.sync_copy(idx_hbm.at[wid], idx_buf)

        def fire(u, r):
            pltpu.async_copy(x_hbm.at[idx_buf.at[pl.ds(u * unit, unit)]],
                             bufs[r], sems[r])

        def drain(r):
            pltpu.make_async_copy(x_hbm.at[pl.ds(0, unit)], bufs[r],
                                  sems[r]).wait()

        for r in range(RING - 1):
            fire(r, r)

        def loop(k, _):
            for r in range(RING):
                u = k * RING + r
                drain(r)
                pltpu.sync_copy(bufs[r],
                                g_hbm.at[pl.ds(base + u * unit, unit)])
                nxt = u + RING - 1

                @pl.when(nxt < per_w)
                def _():
                    fire(nxt, (r + RING - 1) % RING)
            return 0

        lax.fori_loop(0, per_w // RING, loop, 0)
        for e in range(per_w % RING):
            u = (per_w // RING) * RING + e
            drain(e)
            pltpu.sync_copy(bufs[e], g_hbm.at[pl.ds(base + u * unit, unit)])

    mesh = plsc.VectorSubcoreMesh(core_axis_name="c", subcore_axis_name="s",
                                  num_cores=NC, num_subcores=NS)
    return pl.kernel(
        body,
        out_type=jax.ShapeDtypeStruct((NB * nph, D), jnp.float32),
        mesh=mesh,
        compiler_params=pltpu.CompilerParams(use_tc_tiling_on_sc=False),
        scratch_types=(
            [pltpu.VMEM((rows_w,), jnp.int32)]
            + [pltpu.VMEM((unit, D), jnp.float32)] * RING
            + [pltpu.SemaphoreType.DMA] * RING
        ),
    )(x, idx2d)


def _head_body(h2_ref, h1_ref, h0_ref, ws0_ref, wn0_ref, b0_ref,
               ws1_ref, wn1_ref, b1_ref, wp_ref, bp_ref, out_ref):
    blk = out_ref.shape[0]
    f32 = jnp.float32
    ws0 = ws0_ref[...]
    wn0 = wn0_ref[...]
    b0 = b0_ref[...]
    top_self = jnp.dot(h0_ref[0], ws0, preferred_element_type=f32)
    acc = jnp.broadcast_to(bp_ref[...], (blk, D))
    for t in range(T):
        m = h1_ref[t * S1]
        for j in range(1, S1):
            m = m + h1_ref[t * S1 + j]
        mean5 = m / 5.0
        top = top_self + jnp.dot(mean5, wn0, preferred_element_type=f32) + b0
        s0_top = (top >= 1.0).astype(f32)
        m1 = jnp.zeros((blk, H1), f32)
        for j in range(S1):
            mean2 = (h2_ref[(t * S2 + 0) * S1 + j]
                     + h2_ref[(t * S2 + 1) * S1 + j]) / 2.0
            bot = (jnp.dot(h1_ref[t * S1 + j], ws0, preferred_element_type=f32)
                   + jnp.dot(mean2, wn0, preferred_element_type=f32) + b0)
            m1 = m1 + (bot >= 1.0).astype(f32)
        neigh1 = m1 / 5.0
        out1 = (jnp.dot(s0_top, ws1_ref[...], preferred_element_type=f32)
                + jnp.dot(neigh1, wn1_ref[...], preferred_element_type=f32)
                + b1_ref[...])
        s1 = (out1 >= 1.0).astype(f32)
        acc = acc + jnp.dot(s1, wp_ref[t * H2:(t + 1) * H2, :],
                            preferred_element_type=f32)
    out_ref[...] = acc


def _head(g, ws0, wn0, b0, ws1, wn1, b1, wp, bp, nph):
    blk = 256
    grid = nph // blk
    return pl.pallas_call(
        _head_body,
        grid=(grid,),
        in_specs=[
            pl.BlockSpec((T * S2 * S1, blk, D), lambda i: (0, i, 0)),
            pl.BlockSpec((T * S1, blk, D), lambda i: (2, i, 0)),
            pl.BlockSpec((1, blk, D), lambda i: (NB - 1, i, 0)),
            pl.BlockSpec((D, H1), lambda i: (0, 0)),
            pl.BlockSpec((D, H1), lambda i: (0, 0)),
            pl.BlockSpec((1, H1), lambda i: (0, 0)),
            pl.BlockSpec((H1, H2), lambda i: (0, 0)),
            pl.BlockSpec((H1, H2), lambda i: (0, 0)),
            pl.BlockSpec((1, H2), lambda i: (0, 0)),
            pl.BlockSpec((T * H2, D), lambda i: (0, 0)),
            pl.BlockSpec((1, D), lambda i: (0, 0)),
        ],
        out_specs=pl.BlockSpec((blk, D), lambda i: (i, 0)),
        out_shape=jax.ShapeDtypeStruct((nph, D), jnp.float32),
    )(g, g, g, ws0, wn0, b0, ws1, wn1, b1, wp, bp)


def kernel(x, nodes, nbr1, nbr2, W_self0, W_neigh0, b0, W_self1, W_neigh1,
           b1, Wp, bp):
    n = nodes.shape[0]

    idx2 = jnp.zeros((T, S2, S1, NP), jnp.int32)
    idx2 = idx2.at[:, :, :, :n].set(
        nbr2.reshape(T, n, S1, S2).transpose(0, 3, 2, 1))
    idx1 = jnp.zeros((T, S1, NP), jnp.int32)
    idx1 = idx1.at[:, :, :n].set(nbr1.reshape(T, n, S1).transpose(0, 2, 1))
    nodes_pad = jnp.zeros((1, NP), jnp.int32).at[0, :n].set(nodes)
    idx_flat = jnp.concatenate(
        [idx2.reshape(T * S2 * S1, NP), idx1.reshape(T * S1, NP), nodes_pad],
        axis=0)

    outs = []
    off = 0
    for nph in PARTS:
        unit = 128 if (NB * nph // NW) % 128 == 0 else 160
        idx_p = idx_flat[:, off:off + nph].reshape(NW, NB * nph // NW)
        g = _sc_gather(x, idx_p, nph, unit)
        outs.append(_head(g.reshape(NB, nph, D), W_self0, W_neigh0,
                          b0[None, :], W_self1, W_neigh1, b1[None, :],
                          Wp, bp[None, :], nph))
        off += nph
    return jnp.concatenate(outs, axis=0)[:n]
